# baseline (speedup 1.0000x reference)
import functools

import jax
import jax.numpy as jnp
from jax.experimental import pallas as pl
import jax.experimental.pallas.tpu as pltpu

_TOP_K = 8
_NEG = -1e30
_NSTRIPE = 4


def _body(facts_ref, beta_hbm, alT_ref, rs_ref, W_hbm, gamma_ref, lnb_ref,
          out_ref, beta_vmem, W_vmem, bsem, wsem):
    R = beta_hbm.shape[0]
    ch = R // _NSTRIPE
    bcopies, wcopies = [], []
    for c in range(_NSTRIPE):
        sl = pl.ds(c * ch, ch)
        bcopies.append(pltpu.make_async_copy(
            beta_hbm.at[sl, :], beta_vmem.at[sl, :], bsem.at[c]))
        wcopies.append(pltpu.make_async_copy(
            W_hbm.at[sl, :], W_vmem.at[sl, :], wsem.at[c]))
    for cp in bcopies:
        cp.start()
    for cp in wcopies:
        cp.start()

    facts = facts_ref[...]
    B = facts.shape[0]
    log_f = jnp.log(jnp.maximum(facts, 1e-30))
    log_1mf = jnp.log(jnp.maximum(1.0 - facts, 1e-30))
    logs = jnp.concatenate([log_f, log_1mf], axis=0).astype(jnp.bfloat16)

    dn = (((1,), (1,)), ((), ()))
    mm = functools.partial(jax.lax.dot_general, dimension_numbers=dn,
                           preferred_element_type=jnp.float32,
                           precision=jax.lax.Precision.HIGHEST)
    mm_bf = functools.partial(jax.lax.dot_general, dimension_numbers=dn,
                              preferred_element_type=jnp.float32)

    for cp in bcopies:
        cp.wait()
    mask = jnp.where(beta_vmem[...] > 0.0, 1.0, 0.0)
    prods = jnp.exp(mm_bf(logs, mask.astype(jnp.bfloat16)))
    and_agg = prods[:B]
    or_agg = 1.0 - prods[B:]
    s_sum = mm(facts, mask)
    cnt = jnp.sum(mask, axis=1)[None, :] + 1e-08
    k_of_n = s_sum / cnt

    w = jax.nn.softmax(alT_ref[...], axis=0)
    mixed = (w[0][None, :] * and_agg + w[1][None, :] * or_agg
             + w[2][None, :] * k_of_n + w[3][None, :] * (1.0 - k_of_n))
    act = mixed * jax.nn.sigmoid(rs_ref[...])

    iota = jax.lax.broadcasted_iota(jnp.int32, act.shape, 1)
    a = act
    gate = jnp.zeros_like(act)
    for _ in range(_TOP_K):
        m = jnp.max(a, axis=1, keepdims=True)
        idx = jnp.min(jnp.where(a == m, iota, act.shape[1]), axis=1,
                      keepdims=True)
        sel = iota == idx
        gate = jnp.where(sel, 1.0, gate)
        a = jnp.where(sel, _NEG, a)

    for cp in wcopies:
        cp.wait()
    pre = mm(facts, W_vmem[...]) + act * gate
    mu = jnp.mean(pre, axis=1, keepdims=True)
    var = jnp.mean(pre * pre, axis=1, keepdims=True) - mu * mu
    out_ref[...] = ((pre - mu) * jax.lax.rsqrt(var + 1e-05)
                    * gamma_ref[...] + lnb_ref[...])


def kernel(facts, beta, aggregator_logits, rule_strength_raw, W, gamma,
           ln_beta):
    B, D = facts.shape
    R, _ = beta.shape
    return pl.pallas_call(
        _body,
        in_specs=[
            pl.BlockSpec(memory_space=pltpu.MemorySpace.VMEM),
            pl.BlockSpec(memory_space=pltpu.MemorySpace.HBM),
            pl.BlockSpec(memory_space=pltpu.MemorySpace.VMEM),
            pl.BlockSpec(memory_space=pltpu.MemorySpace.VMEM),
            pl.BlockSpec(memory_space=pltpu.MemorySpace.HBM),
            pl.BlockSpec(memory_space=pltpu.MemorySpace.VMEM),
            pl.BlockSpec(memory_space=pltpu.MemorySpace.VMEM),
        ],
        out_specs=pl.BlockSpec(memory_space=pltpu.MemorySpace.VMEM),
        out_shape=jax.ShapeDtypeStruct((B, R), jnp.float32),
        scratch_shapes=[
            pltpu.VMEM((R, D), jnp.float32),
            pltpu.VMEM((R, D), jnp.float32),
            pltpu.SemaphoreType.DMA((_NSTRIPE,)),
            pltpu.SemaphoreType.DMA((_NSTRIPE,)),
        ],
    )(facts, beta, aggregator_logits.T, rule_strength_raw[None, :], W,
      gamma[None, :], ln_beta[None, :])


# speedup vs baseline: 1.3692x; 1.3692x over previous
import functools

import jax
import jax.numpy as jnp
from jax.experimental import pallas as pl

_TOP_K = 8
_NEG = -1e30


def _body(facts_ref, beta_ref, al_ref, rs_ref, W_ref, gamma_ref, lnb_ref,
          out_ref):
    facts = facts_ref[...]
    B = facts.shape[0]
    beta = beta_ref[...]
    mask = jnp.where(beta > 0.0, 1.0, 0.0)
    mask_bf = mask.astype(jnp.bfloat16)

    dn = (((1,), (1,)), ((), ()))
    mm_bf = functools.partial(jax.lax.dot_general, dimension_numbers=dn,
                              preferred_element_type=jnp.float32)

    f_hi = facts.astype(jnp.bfloat16)
    r1 = facts - f_hi.astype(jnp.float32)
    f_lo = r1.astype(jnp.bfloat16)
    f_lo2 = (r1 - f_lo.astype(jnp.float32)).astype(jnp.bfloat16)

    log_f = jnp.log(jnp.maximum(facts, 1e-30))
    log_1mf = jnp.log(jnp.maximum(1.0 - facts, 1e-30))
    big_lhs = jnp.concatenate(
        [log_f.astype(jnp.bfloat16), log_1mf.astype(jnp.bfloat16),
         f_hi, f_lo, f_lo2], axis=0)
    big = mm_bf(big_lhs, mask_bf)
    prods = jnp.exp(big[:2 * B])
    and_agg = prods[:B]
    or_agg = 1.0 - prods[B:]
    s_sum = big[2 * B:3 * B] + big[3 * B:4 * B] + big[4 * B:]
    cnt = jnp.sum(mask, axis=1)[None, :] + 1e-08
    k_of_n = s_sum / cnt

    w = jax.nn.softmax(al_ref[...].T, axis=0)
    mixed = (w[0][None, :] * and_agg + w[1][None, :] * or_agg
             + w[2][None, :] * k_of_n + w[3][None, :] * (1.0 - k_of_n))
    act = mixed * jax.nn.sigmoid(rs_ref[...])

    iota = jax.lax.broadcasted_iota(jnp.int32, act.shape, 1)
    a = act
    gate = jnp.zeros_like(act)
    for _ in range(_TOP_K):
        m = jnp.max(a, axis=1, keepdims=True)
        idx = jnp.min(jnp.where(a == m, iota, act.shape[1]), axis=1,
                      keepdims=True)
        sel = iota == idx
        gate = jnp.where(sel, 1.0, gate)
        a = jnp.where(sel, _NEG, a)

    W_f32 = W_ref[...]
    w_hi = W_f32.astype(jnp.bfloat16)
    w_lo = (W_f32 - w_hi.astype(jnp.float32)).astype(jnp.bfloat16)
    R = W_f32.shape[0]
    P = mm_bf(jnp.concatenate([f_hi, f_lo], axis=0),
              jnp.concatenate([w_hi, w_lo], axis=0))
    proj = (P[:B, :R] + P[:B, R:]) + (P[B:, :R] + P[B:, R:])
    pre = proj + act * gate
    mu = jnp.mean(pre, axis=1, keepdims=True)
    var = jnp.mean(pre * pre, axis=1, keepdims=True) - mu * mu
    out_ref[...] = ((pre - mu) * jax.lax.rsqrt(var + 1e-05)
                    * gamma_ref[...] + lnb_ref[...])


def kernel(facts, beta, aggregator_logits, rule_strength_raw, W, gamma,
           ln_beta):
    B, _ = facts.shape
    R, _ = beta.shape
    return pl.pallas_call(
        _body,
        out_shape=jax.ShapeDtypeStruct((B, R), jnp.float32),
    )(facts, beta, aggregator_logits, rule_strength_raw[None, :], W,
      gamma[None, :], ln_beta[None, :])
